# Initial kernel scaffold; baseline (speedup 1.0000x reference)
import functools
import jax
import jax.numpy as jnp
from jax import lax
from jax.experimental import pallas as pl
from jax.experimental.pallas import tpu as pltpu

N = 320000
D = 128
H = 256
S = 10000

B = 1280
NB = N // B
W = 128
S_PAD = 10240


def _pointnet_segsum_body(x_ref, idx_ref, w1_ref, b1_ref, w2_ref, b2_ref,
                          acc_ref):
    i = pl.program_id(0)

    @pl.when(i == 0)
    def _():
        acc_ref[...] = jnp.zeros_like(acc_ref)

    x = x_ref[...]
    h = jnp.dot(x.astype(jnp.bfloat16), w1_ref[...],
                preferred_element_type=jnp.float32)
    h = jnp.maximum(h + b1_ref[...], 0.0)
    pt = jnp.dot(h.astype(jnp.bfloat16), w2_ref[...],
                 preferred_element_type=jnp.float32)
    pt = pt + b2_ref[...]
    pt_bf = pt.astype(jnp.bfloat16)

    idxv = idx_ref[0, 0, :]
    first = jnp.min(idxv)
    last = jnp.max(idxv)
    w0 = (first // W) * W
    nwin = (last // W) - (first // W) + 1

    def body(c, carry):
        ws = pl.multiple_of(w0 + c * W, W)
        seg_ids = ws + lax.broadcasted_iota(jnp.int32, (W, B), 0)
        oh = (seg_ids == idxv[None, :]).astype(jnp.bfloat16)
        contrib = lax.dot_general(oh, pt_bf, (((1,), (0,)), ((), ())),
                                  preferred_element_type=jnp.float32)
        acc_ref[pl.ds(ws, W), :] += contrib
        return carry

    lax.fori_loop(0, nwin, body, 0)


def _reduce_net_body(seg_ref, w1_ref, b1_ref, w2_ref, b2_ref, out_ref):
    seg = seg_ref[...]
    h = jnp.dot(seg.astype(jnp.bfloat16), w1_ref[...],
                preferred_element_type=jnp.float32)
    h = jnp.maximum(h + b1_ref[...], 0.0)
    out = jnp.dot(h.astype(jnp.bfloat16), w2_ref[...],
                  preferred_element_type=jnp.float32)
    out_ref[...] = out + b2_ref[...]


def kernel(x, idx, W1p, b1p, W2p, b2p, W1r, b1r, W2r, b2r):
    idx3 = idx.astype(jnp.int32).reshape(NB, 1, B)
    w1p_bf = W1p.astype(jnp.bfloat16)
    w2p_bf = W2p.astype(jnp.bfloat16)
    w1r_bf = W1r.astype(jnp.bfloat16)
    w2r_bf = W2r.astype(jnp.bfloat16)
    b1p2 = b1p.reshape(1, H)
    b2p2 = b2p.reshape(1, D)
    b1r2 = b1r.reshape(1, H)
    b2r2 = b2r.reshape(1, D)

    seg = pl.pallas_call(
        _pointnet_segsum_body,
        grid=(NB,),
        in_specs=[
            pl.BlockSpec((B, D), lambda i: (i, 0)),
            pl.BlockSpec((1, 1, B), lambda i: (i, 0, 0)),
            pl.BlockSpec((D, H), lambda i: (0, 0)),
            pl.BlockSpec((1, H), lambda i: (0, 0)),
            pl.BlockSpec((H, D), lambda i: (0, 0)),
            pl.BlockSpec((1, D), lambda i: (0, 0)),
        ],
        out_specs=pl.BlockSpec((S_PAD, D), lambda i: (0, 0)),
        out_shape=jax.ShapeDtypeStruct((S_PAD, D), jnp.float32),
    )(x, idx3, w1p_bf, b1p2, w2p_bf, b2p2)

    out = pl.pallas_call(
        _reduce_net_body,
        grid=(S_PAD // 1024,),
        in_specs=[
            pl.BlockSpec((1024, D), lambda i: (i, 0)),
            pl.BlockSpec((D, H), lambda i: (0, 0)),
            pl.BlockSpec((1, H), lambda i: (0, 0)),
            pl.BlockSpec((H, D), lambda i: (0, 0)),
            pl.BlockSpec((1, D), lambda i: (0, 0)),
        ],
        out_specs=pl.BlockSpec((1024, D), lambda i: (i, 0)),
        out_shape=jax.ShapeDtypeStruct((S_PAD, D), jnp.float32),
    )(seg, w1r_bf, b1r2, w2r_bf, b2r2)

    return out[:S]


# speedup vs baseline: 3.8743x; 3.8743x over previous
import functools
import jax
import jax.numpy as jnp
from jax import lax
from jax.experimental import pallas as pl
from jax.experimental.pallas import tpu as pltpu

N = 320000
D = 128
H = 256
S = 10000

B = 1280
NB = N // B
W = 128
S_PAD = 10240


def _pointnet_segsum_body(x_ref, idx_ref, w1_ref, b1_ref, w2_ref, b2_ref,
                          acc_ref):
    i = pl.program_id(0)

    @pl.when(i == 0)
    def _():
        acc_ref[...] = jnp.zeros_like(acc_ref)

    x = x_ref[...]
    h = jnp.dot(x.astype(jnp.bfloat16), w1_ref[...],
                preferred_element_type=jnp.float32)
    h = jnp.maximum(h + b1_ref[...], 0.0)
    pt = jnp.dot(h.astype(jnp.bfloat16), w2_ref[...],
                 preferred_element_type=jnp.float32)
    pt = pt + b2_ref[...]
    pt_bf = pt.astype(jnp.bfloat16)

    idxv = idx_ref[0, 0, :]
    nb = idxv.shape[0]
    first = jnp.min(idxv)
    last = jnp.max(idxv)
    w0 = (first // W) * W
    nwin = (last // W) - (first // W) + 1

    def body(c, carry):
        ws = pl.multiple_of(w0 + c * W, W)
        seg_ids = ws + lax.broadcasted_iota(jnp.int32, (W, nb), 0)
        oh = (seg_ids == idxv[None, :]).astype(jnp.bfloat16)
        contrib = lax.dot_general(oh, pt_bf, (((1,), (0,)), ((), ())),
                                  preferred_element_type=jnp.float32)
        acc_ref[pl.ds(ws, W), :] += contrib
        return carry

    lax.fori_loop(0, nwin, body, 0)


def _reduce_net_body(seg_ref, w1_ref, b1_ref, w2_ref, b2_ref, out_ref):
    seg = seg_ref[...]
    h = jnp.dot(seg.astype(jnp.bfloat16), w1_ref[...],
                preferred_element_type=jnp.float32)
    h = jnp.maximum(h + b1_ref[...], 0.0)
    out = jnp.dot(h.astype(jnp.bfloat16), w2_ref[...],
                  preferred_element_type=jnp.float32)
    out_ref[...] = out + b2_ref[...]


def kernel(x, idx, W1p, b1p, W2p, b2p, W1r, b1r, W2r, b2r):
    idx3 = idx.astype(jnp.int32).reshape(NB, 1, B)
    w1p_bf = W1p.astype(jnp.bfloat16)
    w2p_bf = W2p.astype(jnp.bfloat16)
    w1r_bf = W1r.astype(jnp.bfloat16)
    w2r_bf = W2r.astype(jnp.bfloat16)
    b1p2 = b1p.reshape(1, H)
    b2p2 = b2p.reshape(1, D)
    b1r2 = b1r.reshape(1, H)
    b2r2 = b2r.reshape(1, D)

    seg = pl.pallas_call(
        _pointnet_segsum_body,
        grid=(NB,),
        in_specs=[
            pl.BlockSpec((B, D), lambda i: (i, 0)),
            pl.BlockSpec((1, 1, B), lambda i: (i, 0, 0)),
            pl.BlockSpec((D, H), lambda i: (0, 0)),
            pl.BlockSpec((1, H), lambda i: (0, 0)),
            pl.BlockSpec((H, D), lambda i: (0, 0)),
            pl.BlockSpec((1, D), lambda i: (0, 0)),
        ],
        out_specs=pl.BlockSpec((S_PAD, D), lambda i: (0, 0)),
        out_shape=jax.ShapeDtypeStruct((S_PAD, D), jnp.float32),
    )(x, idx3, w1p_bf, b1p2, w2p_bf, b2p2)

    out = pl.pallas_call(
        _reduce_net_body,
        grid=(S_PAD // 1024,),
        in_specs=[
            pl.BlockSpec((1024, D), lambda i: (i, 0)),
            pl.BlockSpec((D, H), lambda i: (0, 0)),
            pl.BlockSpec((1, H), lambda i: (0, 0)),
            pl.BlockSpec((H, D), lambda i: (0, 0)),
            pl.BlockSpec((1, D), lambda i: (0, 0)),
        ],
        out_specs=pl.BlockSpec((1024, D), lambda i: (i, 0)),
        out_shape=jax.ShapeDtypeStruct((S_PAD, D), jnp.float32),
    )(seg, w1r_bf, b1r2, w2r_bf, b2r2)

    return out[:S]
